# baseline (speedup 1.0000x reference)
import functools

import jax
import jax.numpy as jnp
from jax import lax
from jax.experimental import pallas as pl
from jax.experimental.pallas import tpu as pltpu
from jax.experimental.pallas import tpu_sc as plsc

TM = 256
TOKENS_PER_W = 256
CHUNK = 32


def _route_body(x_ref, wg_ref, bg_ref, pos_ref, te_ref, logits_s):
    i = pl.program_id(0)
    nt = pl.num_programs(0)
    tm = x_ref.shape[0]
    logits_s[pl.ds(i * tm, tm), :] = (
        lax.dot_general(
            x_ref[...], wg_ref[...], (((1,), (0,)), ((), ())),
            preferred_element_type=jnp.float32,
            precision=lax.Precision.DEFAULT,
        )
        + bg_ref[...]
    )

    @pl.when(i == nt - 1)
    def _route():
        logits = logits_s[...]
        t, e = logits.shape
        ecol = lax.broadcasted_iota(jnp.int32, (t, e), 1)
        m1 = jnp.max(logits, axis=1, keepdims=True)
        idx1 = jnp.min(jnp.where(logits == m1, ecol, e), axis=1, keepdims=True)
        mask1 = ecol == idx1
        l2 = jnp.where(mask1, -jnp.inf, logits)
        m2 = jnp.max(l2, axis=1, keepdims=True)
        idx2 = jnp.min(jnp.where(l2 == m2, ecol, e), axis=1, keepdims=True)
        mask2 = ecol == idx2

        oh = jnp.concatenate(
            [mask1.astype(jnp.int32), mask2.astype(jnp.int32)], axis=1)
        c = oh
        s = 1
        while s < t:
            sh = jnp.concatenate(
                [jnp.zeros((s, 2 * e), jnp.int32), c[: t - s, :]], axis=0)
            c = c + sh
            s *= 2
        ex = c - oh
        last = c[t - 1, :]

        starts = []
        cnt1 = []
        run = jnp.int32(0)
        for j in range(e):
            starts.append(run)
            c1 = last[j]
            cnt1.append(c1)
            tot = c1 + last[e + j]
            run = ((run + tot + TM - 1) // TM) * TM

        lane2e = lax.broadcasted_iota(jnp.int32, (1, 2 * e), 1)
        add = jnp.zeros((1, 2 * e), jnp.int32)
        for j in range(e):
            add = add + jnp.where(lane2e == j, starts[j], 0)
            add = add + jnp.where(lane2e == e + j, starts[j] + cnt1[j], 0)

        posb = oh * (ex + add)
        pos1 = jnp.sum(jnp.where(lane2e < e, posb, 0), axis=1, keepdims=True)
        pos2 = jnp.sum(jnp.where(lane2e >= e, posb, 0), axis=1, keepdims=True)
        ecol8 = lax.broadcasted_iota(jnp.int32, (t, e), 1)
        pos_ref[...] = jnp.where(ecol8 == 0, pos1, 0) + jnp.where(
            ecol8 == 1, pos2, 0)

        tcol = lax.broadcasted_iota(jnp.int32, te_ref.shape, 1)
        te = jnp.zeros(te_ref.shape, jnp.int32)
        for j in range(1, e):
            te = te + jnp.where(tcol * TM >= starts[j], 1, 0)
        te_ref[...] = te


def _dispatch_body(xt_hbm, p1_hbm, p2_hbm, xs_hbm, i1_v, i2_v, rows_v, sem):
    w = lax.axis_index("s") * 2 + lax.axis_index("c")
    pltpu.sync_copy(p1_hbm.at[w], i1_v)
    pltpu.sync_copy(p2_hbm.at[w], i2_v)
    base = w * TOKENS_PER_W
    for g in range(TOKENS_PER_W // CHUNK):
        pltpu.sync_copy(xt_hbm.at[pl.ds(base + g * CHUNK, CHUNK)], rows_v)
        c1 = pltpu.async_copy(rows_v, xs_hbm.at[i1_v.at[g]], sem)
        c2 = pltpu.async_copy(rows_v, xs_hbm.at[i2_v.at[g]], sem)
        c1.wait()
        c2.wait()


def _gmm_body(te_ref, xs_ref, we_ref, be_ref, o_ref):
    del te_ref
    y = lax.dot_general(
        xs_ref[...], we_ref[0], (((1,), (0,)), ((), ())),
        preferred_element_type=jnp.float32,
        precision=lax.Precision.DEFAULT,
    )
    o_ref[...] = jnp.maximum(y + be_ref[0], 0.0)


def _combine_body(rows_hbm, p1_hbm, p2_hbm, o_hbm, i1_v, i2_v, b1_v, b2_v, sem):
    w = lax.axis_index("s") * 2 + lax.axis_index("c")
    pltpu.sync_copy(p1_hbm.at[w], i1_v)
    pltpu.sync_copy(p2_hbm.at[w], i2_v)
    base = w * TOKENS_PER_W
    d = b1_v.shape[1]
    for g in range(TOKENS_PER_W // CHUNK):
        c1 = pltpu.async_copy(rows_hbm.at[i1_v.at[g]], b1_v, sem)
        c2 = pltpu.async_copy(rows_hbm.at[i2_v.at[g]], b2_v, sem)
        c1.wait()
        c2.wait()

        def body(it, _):
            r = it // (d // 16)
            cc = (it % (d // 16)) * 16
            b1_v[r, pl.ds(cc, 16)] = (
                b1_v[r, pl.ds(cc, 16)] + b2_v[r, pl.ds(cc, 16)]) * 0.5
            return 0

        lax.fori_loop(0, CHUNK * (d // 16), body, 0)
        pltpu.sync_copy(b1_v, o_hbm.at[pl.ds(base + g * CHUNK, CHUNK)])


@functools.partial(jax.jit, static_argnums=())
def kernel(x, Wg, bg, We, be):
    n, s, v = x.shape
    e = Wg.shape[1]
    out = We.shape[2]
    t = n * s
    xt = x.reshape(t, v)
    tm_g = 1024
    nt_g = t // tm_g

    rows_pad = ((2 * t + e * (TM - 1)) // TM + 1) * TM
    n_tiles = rows_pad // TM

    posout, te = pl.pallas_call(
        _route_body,
        grid=(nt_g,),
        in_specs=[
            pl.BlockSpec((tm_g, v), lambda i: (i, 0)),
            pl.BlockSpec((v, e), lambda i: (0, 0)),
            pl.BlockSpec((1, e), lambda i: (0, 0)),
        ],
        out_specs=[
            pl.BlockSpec((t, e), lambda i: (0, 0)),
            pl.BlockSpec((1, 128), lambda i: (0, 0)),
        ],
        out_shape=[
            jax.ShapeDtypeStruct((t, e), jnp.int32),
            jax.ShapeDtypeStruct((1, 128), jnp.int32),
        ],
        scratch_shapes=[pltpu.VMEM((t, e), jnp.float32)],
    )(xt, Wg, bg.reshape(1, e))

    nw = 32
    nch = TOKENS_PER_W // CHUNK
    pos1 = posout[:, 0].reshape(nw, nch, CHUNK)
    pos2 = posout[:, 1].reshape(nw, nch, CHUNK)
    te_flat = te.reshape(128)[:n_tiles]

    mesh = plsc.VectorSubcoreMesh(core_axis_name="c", subcore_axis_name="s")

    xs = pl.kernel(
        _dispatch_body,
        out_type=jax.ShapeDtypeStruct((rows_pad, v), jnp.float32),
        mesh=mesh,
        scratch_types=[
            pltpu.VMEM((nch, CHUNK), jnp.int32),
            pltpu.VMEM((nch, CHUNK), jnp.int32),
            pltpu.VMEM((CHUNK, v), jnp.float32),
            pltpu.SemaphoreType.DMA,
        ],
    )(xt, pos1, pos2)

    orows = pl.pallas_call(
        _gmm_body,
        grid_spec=pltpu.PrefetchScalarGridSpec(
            num_scalar_prefetch=1,
            grid=(n_tiles,),
            in_specs=[
                pl.BlockSpec((TM, v), lambda i, te_r: (i, 0)),
                pl.BlockSpec((1, v, out), lambda i, te_r: (te_r[i], 0, 0)),
                pl.BlockSpec((1, 1, out), lambda i, te_r: (te_r[i], 0, 0)),
            ],
            out_specs=pl.BlockSpec((TM, out), lambda i, te_r: (i, 0)),
        ),
        out_shape=jax.ShapeDtypeStruct((rows_pad, out), jnp.float32),
    )(te_flat, xs, We, be.reshape(e, 1, out))

    o = pl.kernel(
        _combine_body,
        out_type=jax.ShapeDtypeStruct((t, out), jnp.float32),
        mesh=mesh,
        scratch_types=[
            pltpu.VMEM((nch, CHUNK), jnp.int32),
            pltpu.VMEM((nch, CHUNK), jnp.int32),
            pltpu.VMEM((CHUNK, out), jnp.float32),
            pltpu.VMEM((CHUNK, out), jnp.float32),
            pltpu.SemaphoreType.DMA,
        ],
    )(orows, pos1, pos2)

    return o.reshape(n, s, out)


# speedup vs baseline: 1.2661x; 1.2661x over previous
import functools

import jax
import jax.numpy as jnp
from jax import lax
from jax.experimental import pallas as pl
from jax.experimental.pallas import tpu as pltpu
from jax.experimental.pallas import tpu_sc as plsc

TM = 256
TOKENS_PER_W = 256
CHUNK = 32


def _route_body(x_ref, wg_ref, bg_ref, pos_ref, te_ref, logits_s):
    i = pl.program_id(0)
    nt = pl.num_programs(0)
    tm = x_ref.shape[0]
    logits_s[pl.ds(i * tm, tm), :] = (
        lax.dot_general(
            x_ref[...], wg_ref[...], (((1,), (0,)), ((), ())),
            preferred_element_type=jnp.float32,
            precision=lax.Precision.DEFAULT,
        )
        + bg_ref[...]
    )

    @pl.when(i == nt - 1)
    def _route():
        logits = logits_s[...]
        t, e = logits.shape
        ecol = lax.broadcasted_iota(jnp.int32, (t, e), 1)
        m1 = jnp.max(logits, axis=1, keepdims=True)
        idx1 = jnp.min(jnp.where(logits == m1, ecol, e), axis=1, keepdims=True)
        mask1 = ecol == idx1
        l2 = jnp.where(mask1, -jnp.inf, logits)
        m2 = jnp.max(l2, axis=1, keepdims=True)
        idx2 = jnp.min(jnp.where(l2 == m2, ecol, e), axis=1, keepdims=True)
        mask2 = ecol == idx2

        oh = jnp.concatenate(
            [mask1.astype(jnp.int32), mask2.astype(jnp.int32)], axis=1)
        c = oh
        s = 1
        while s < t:
            sh = jnp.concatenate(
                [jnp.zeros((s, 2 * e), jnp.int32), c[: t - s, :]], axis=0)
            c = c + sh
            s *= 2
        ex = c - oh
        last = c[t - 1, :]

        starts = []
        cnt1 = []
        run = jnp.int32(0)
        for j in range(e):
            starts.append(run)
            c1 = last[j]
            cnt1.append(c1)
            tot = c1 + last[e + j]
            run = ((run + tot + TM - 1) // TM) * TM

        lane2e = lax.broadcasted_iota(jnp.int32, (1, 2 * e), 1)
        add = jnp.zeros((1, 2 * e), jnp.int32)
        for j in range(e):
            add = add + jnp.where(lane2e == j, starts[j], 0)
            add = add + jnp.where(lane2e == e + j, starts[j] + cnt1[j], 0)

        posb = oh * (ex + add)
        pos1 = jnp.sum(jnp.where(lane2e < e, posb, 0), axis=1, keepdims=True)
        pos2 = jnp.sum(jnp.where(lane2e >= e, posb, 0), axis=1, keepdims=True)
        ecol8 = lax.broadcasted_iota(jnp.int32, (t, e), 1)
        pos_ref[...] = jnp.where(ecol8 == 0, pos1, 0) + jnp.where(
            ecol8 == 1, pos2, 0)

        tcol = lax.broadcasted_iota(jnp.int32, te_ref.shape, 1)
        te = jnp.zeros(te_ref.shape, jnp.int32)
        for j in range(1, e):
            te = te + jnp.where(tcol * TM >= starts[j], 1, 0)
        te_ref[...] = te


def _dispatch_body(xt_hbm, p1_hbm, p2_hbm, xs_hbm, i1_v, i2_v, rows_v, sem):
    w = lax.axis_index("s") * 2 + lax.axis_index("c")
    pltpu.sync_copy(p1_hbm.at[w], i1_v)
    pltpu.sync_copy(p2_hbm.at[w], i2_v)
    base = w * TOKENS_PER_W
    for g in range(TOKENS_PER_W // CHUNK):
        pltpu.sync_copy(xt_hbm.at[pl.ds(base + g * CHUNK, CHUNK)], rows_v)
        c1 = pltpu.async_copy(rows_v, xs_hbm.at[i1_v.at[g]], sem)
        c2 = pltpu.async_copy(rows_v, xs_hbm.at[i2_v.at[g]], sem)
        c1.wait()
        c2.wait()


def _gmm_body(te_ref, xs_ref, we_ref, be_ref, o_ref):
    del te_ref
    y = lax.dot_general(
        xs_ref[...], we_ref[0], (((1,), (0,)), ((), ())),
        preferred_element_type=jnp.float32,
        precision=lax.Precision.DEFAULT,
    )
    o_ref[...] = jnp.maximum(y + be_ref[0], 0.0)


def _combine_body(rows_hbm, p1_hbm, p2_hbm, o_hbm, i1_v, i2_v, b1_v, b2_v, sem):
    w = lax.axis_index("s") * 2 + lax.axis_index("c")
    pltpu.sync_copy(p1_hbm.at[w], i1_v)
    pltpu.sync_copy(p2_hbm.at[w], i2_v)
    base = w * TOKENS_PER_W
    d = b1_v.shape[1]

    def gbody(g, _):
        c1 = pltpu.async_copy(rows_hbm.at[i1_v.at[g]], b1_v, sem)
        c2 = pltpu.async_copy(rows_hbm.at[i2_v.at[g]], b2_v, sem)
        c1.wait()
        c2.wait()

        for r in range(CHUNK):
            @plsc.parallel_loop(0, d, step=16, unroll=8)
            def _avg(cc, r=r):
                b1_v[r, pl.ds(cc, 16)] = (
                    b1_v[r, pl.ds(cc, 16)] + b2_v[r, pl.ds(cc, 16)]) * 0.5

        pltpu.sync_copy(b1_v, o_hbm.at[pl.ds(base + g * CHUNK, CHUNK)])
        return 0

    lax.fori_loop(0, TOKENS_PER_W // CHUNK, gbody, 0)


@functools.partial(jax.jit, static_argnums=())
def kernel(x, Wg, bg, We, be):
    n, s, v = x.shape
    e = Wg.shape[1]
    out = We.shape[2]
    t = n * s
    xt = x.reshape(t, v)
    tm_g = 1024
    nt_g = t // tm_g

    rows_pad = ((2 * t + e * (TM - 1)) // TM + 1) * TM
    n_tiles = rows_pad // TM

    posout, te = pl.pallas_call(
        _route_body,
        grid=(nt_g,),
        in_specs=[
            pl.BlockSpec((tm_g, v), lambda i: (i, 0)),
            pl.BlockSpec((v, e), lambda i: (0, 0)),
            pl.BlockSpec((1, e), lambda i: (0, 0)),
        ],
        out_specs=[
            pl.BlockSpec((t, e), lambda i: (0, 0)),
            pl.BlockSpec((1, 128), lambda i: (0, 0)),
        ],
        out_shape=[
            jax.ShapeDtypeStruct((t, e), jnp.int32),
            jax.ShapeDtypeStruct((1, 128), jnp.int32),
        ],
        scratch_shapes=[pltpu.VMEM((t, e), jnp.float32)],
    )(xt, Wg, bg.reshape(1, e))

    nw = 32
    nch = TOKENS_PER_W // CHUNK
    pos1 = posout[:, 0].reshape(nw, nch, CHUNK)
    pos2 = posout[:, 1].reshape(nw, nch, CHUNK)
    te_flat = te.reshape(128)[:n_tiles]

    mesh = plsc.VectorSubcoreMesh(core_axis_name="c", subcore_axis_name="s")

    xs = pl.kernel(
        _dispatch_body,
        out_type=jax.ShapeDtypeStruct((rows_pad, v), jnp.float32),
        mesh=mesh,
        scratch_types=[
            pltpu.VMEM((nch, CHUNK), jnp.int32),
            pltpu.VMEM((nch, CHUNK), jnp.int32),
            pltpu.VMEM((CHUNK, v), jnp.float32),
            pltpu.SemaphoreType.DMA,
        ],
    )(xt, pos1, pos2)

    orows = pl.pallas_call(
        _gmm_body,
        grid_spec=pltpu.PrefetchScalarGridSpec(
            num_scalar_prefetch=1,
            grid=(n_tiles,),
            in_specs=[
                pl.BlockSpec((TM, v), lambda i, te_r: (i, 0)),
                pl.BlockSpec((1, v, out), lambda i, te_r: (te_r[i], 0, 0)),
                pl.BlockSpec((1, 1, out), lambda i, te_r: (te_r[i], 0, 0)),
            ],
            out_specs=pl.BlockSpec((TM, out), lambda i, te_r: (i, 0)),
        ),
        out_shape=jax.ShapeDtypeStruct((rows_pad, out), jnp.float32),
    )(te_flat, xs, We, be.reshape(e, 1, out))

    o = pl.kernel(
        _combine_body,
        out_type=jax.ShapeDtypeStruct((t, out), jnp.float32),
        mesh=mesh,
        scratch_types=[
            pltpu.VMEM((nch, CHUNK), jnp.int32),
            pltpu.VMEM((nch, CHUNK), jnp.int32),
            pltpu.VMEM((CHUNK, out), jnp.float32),
            pltpu.VMEM((CHUNK, out), jnp.float32),
            pltpu.SemaphoreType.DMA,
        ],
    )(orows, pos1, pos2)

    return o.reshape(n, s, out)
